# baseline (speedup 1.0000x reference)
import jax
import jax.numpy as jnp
from jax import lax
from jax.experimental import pallas as pl
from jax.experimental.pallas import tpu as pltpu
from jax.experimental.pallas import tpu_sc as plsc

NUM_EXPERTS = 64
LANES = 16
NUM_WORKERS = 32


TSTRIDE = 513


def _sc_body(xt_hbm, hist_out, csum_out, chunk, trans, hist, csum, dma_sem):
  nc = 2
  wid = lax.axis_index("s") * nc + lax.axis_index("c")
  rows = chunk.shape[1]
  tile_r, tile_c = 8, 128

  copies = [
      pltpu.make_async_copy(
          xt_hbm.at[pl.ds(i * tile_r, tile_r), pl.ds(wid * rows, rows)],
          chunk.at[pl.ds(i * tile_r, tile_r), :],
          dma_sem,
      )
      for i in range(NUM_EXPERTS // tile_r)
  ]
  for c in copies:
    c.start()

  zf = jnp.zeros((LANES,), jnp.float32)
  for i in range(NUM_EXPERTS):
    hist[pl.ds(i * LANES, LANES)] = zf

  for c in copies:
    c.wait()

  lane = lax.iota(jnp.int32, LANES)
  lane64 = lane * NUM_EXPERTS
  ones = jnp.ones((LANES,), jnp.float32)

  cview_len = tile_r * TSTRIDE

  def repack(i, carry):
    cv = chunk.at[pl.ds(i * tile_r, tile_r), :]
    tv = trans.at[pl.ds(i * cview_len, cview_len)]
    sv = csum.at[pl.ds(i * tile_r, tile_r), :]
    nk = tile_c // LANES
    for cc in range(tile_r):
      acc = None
      for j in range(rows // tile_c):
        vs = [cv[cc, pl.ds(j * tile_c + k * LANES, LANES)] for k in range(nk)]
        for k in range(nk):
          acc = vs[k] if acc is None else acc + vs[k]
          tv[pl.ds(cc * TSTRIDE + j * tile_c + k * LANES, LANES)] = vs[k]
      sv[cc, pl.ds(0, LANES)] = acc
    return carry

  lax.fori_loop(0, NUM_EXPERTS // tile_r, repack, 0)
  tlen2 = trans.shape[0] - (rows - 2 * LANES)

  zi = jnp.zeros((LANES,), jnp.int32)

  def pair(p, carry):
    view = trans.at[pl.ds(p * (2 * LANES), tlen2)]
    states = [[q, view[pl.ds(q * LANES, LANES)], zi] for q in range(2)]
    for t in range(1, NUM_EXPERTS):
      for st in states:
        v = view[pl.ds(t * TSTRIDE + st[0] * LANES, LANES)]
        gt = v > st[1]
        st[1] = jnp.where(gt, v, st[1])
        st[2] = jnp.where(gt, jnp.int32(t), st[2])
    for _, _, am in states:
      plsc.addupdate_scatter(hist, [lane64 + am], ones)
    return carry

  lax.fori_loop(0, rows // (2 * LANES), pair, 0)

  pltpu.sync_copy(hist, hist_out.at[wid])
  pltpu.sync_copy(csum, csum_out.at[wid])


def _finish_body(h_ref, c_ref, o_ref, *, scale):
  h = jnp.sum(h_ref[...], axis=0)
  c = jnp.sum(jnp.sum(c_ref[...], axis=0), axis=-1)
  c_rep = jnp.tile(c, LANES)
  o_ref[0, 0] = scale * jnp.sum(h * c_rep)


def kernel(expert_probs):
  b, e = expert_probs.shape
  rows = b // NUM_WORKERS

  sc_part = pl.kernel(
      _sc_body,
      out_type=[
          jax.ShapeDtypeStruct((NUM_WORKERS, LANES * NUM_EXPERTS), jnp.float32),
          jax.ShapeDtypeStruct((NUM_WORKERS, NUM_EXPERTS, LANES), jnp.float32),
      ],
      mesh=plsc.VectorSubcoreMesh(core_axis_name="c", subcore_axis_name="s"),
      compiler_params=pltpu.CompilerParams(needs_layout_passes=False),
      scratch_types=[
          pltpu.VMEM((NUM_EXPERTS, rows), jnp.float32),
          pltpu.VMEM((NUM_EXPERTS * TSTRIDE,), jnp.float32),
          pltpu.VMEM((LANES * NUM_EXPERTS,), jnp.float32),
          pltpu.VMEM((NUM_EXPERTS, LANES), jnp.float32),
          pltpu.SemaphoreType.DMA,
      ],
  )

  hist, csum = sc_part(expert_probs.T)

  scale = float(e) / (float(b) * float(b))
  finish = pl.pallas_call(
      lambda h, c, o: _finish_body(h, c, o, scale=scale),
      out_shape=jax.ShapeDtypeStruct((1, 1), jnp.float32),
      out_specs=pl.BlockSpec(memory_space=pltpu.SMEM),
  )
  dot = finish(hist, csum)
  return dot[0, 0]


# speedup vs baseline: 1.0066x; 1.0066x over previous
import jax
import jax.numpy as jnp
from jax import lax
from jax.experimental import pallas as pl
from jax.experimental.pallas import tpu as pltpu
from jax.experimental.pallas import tpu_sc as plsc

NUM_EXPERTS = 64
LANES = 16
NUM_WORKERS = 32


TSTRIDE = 513


def _sc_body(xt_hbm, hist_out, csum_out, chunk, trans, hist, csum):
  nc = 2
  wid = lax.axis_index("s") * nc + lax.axis_index("c")
  rows = chunk.shape[1]
  tile_r, tile_c = 8, 128

  pltpu.sync_copy(xt_hbm.at[:, pl.ds(wid * rows, rows)], chunk)

  zf = jnp.zeros((LANES,), jnp.float32)
  for i in range(NUM_EXPERTS):
    hist[pl.ds(i * LANES, LANES)] = zf

  lane = lax.iota(jnp.int32, LANES)
  lane64 = lane * NUM_EXPERTS
  ones = jnp.ones((LANES,), jnp.float32)

  cview_len = tile_r * TSTRIDE

  def repack(i, carry):
    cv = chunk.at[pl.ds(i * tile_r, tile_r), :]
    tv = trans.at[pl.ds(i * cview_len, cview_len)]
    sv = csum.at[pl.ds(i * tile_r, tile_r), :]
    nk = tile_c // LANES
    for cc in range(tile_r):
      acc = None
      for j in range(rows // tile_c):
        vs = [cv[cc, pl.ds(j * tile_c + k * LANES, LANES)] for k in range(nk)]
        for k in range(nk):
          acc = vs[k] if acc is None else acc + vs[k]
          tv[pl.ds(cc * TSTRIDE + j * tile_c + k * LANES, LANES)] = vs[k]
      sv[cc, pl.ds(0, LANES)] = acc
    return carry

  lax.fori_loop(0, NUM_EXPERTS // tile_r, repack, 0)
  tlen2 = trans.shape[0] - (rows - 2 * LANES)

  zi = jnp.zeros((LANES,), jnp.int32)

  def pair(p, carry):
    view = trans.at[pl.ds(p * (2 * LANES), tlen2)]
    states = [[q, view[pl.ds(q * LANES, LANES)], zi] for q in range(2)]
    for t in range(1, NUM_EXPERTS):
      for st in states:
        v = view[pl.ds(t * TSTRIDE + st[0] * LANES, LANES)]
        gt = v > st[1]
        st[1] = jnp.where(gt, v, st[1])
        st[2] = jnp.where(gt, jnp.int32(t), st[2])
    for _, _, am in states:
      plsc.addupdate_scatter(hist, [lane64 + am], ones)
    return carry

  lax.fori_loop(0, rows // (2 * LANES), pair, 0)

  pltpu.sync_copy(hist, hist_out.at[wid])
  pltpu.sync_copy(csum, csum_out.at[wid])


def _finish_body(h_ref, c_ref, o_ref, *, scale):
  h = jnp.sum(h_ref[...], axis=0)
  c = jnp.sum(jnp.sum(c_ref[...], axis=0), axis=-1)
  c_rep = jnp.tile(c, LANES)
  o_ref[0, 0] = scale * jnp.sum(h * c_rep)


def kernel(expert_probs):
  b, e = expert_probs.shape
  rows = b // NUM_WORKERS

  sc_part = pl.kernel(
      _sc_body,
      out_type=[
          jax.ShapeDtypeStruct((NUM_WORKERS, LANES * NUM_EXPERTS), jnp.float32),
          jax.ShapeDtypeStruct((NUM_WORKERS, NUM_EXPERTS, LANES), jnp.float32),
      ],
      mesh=plsc.VectorSubcoreMesh(core_axis_name="c", subcore_axis_name="s"),
      compiler_params=pltpu.CompilerParams(needs_layout_passes=False),
      scratch_types=[
          pltpu.VMEM((NUM_EXPERTS, rows), jnp.float32),
          pltpu.VMEM((NUM_EXPERTS * TSTRIDE,), jnp.float32),
          pltpu.VMEM((LANES * NUM_EXPERTS,), jnp.float32),
          pltpu.VMEM((NUM_EXPERTS, LANES), jnp.float32),
      ],
  )

  hist, csum = sc_part(expert_probs.T)

  scale = float(e) / (float(b) * float(b))
  finish = pl.pallas_call(
      lambda h, c, o: _finish_body(h, c, o, scale=scale),
      out_shape=jax.ShapeDtypeStruct((1, 1), jnp.float32),
      out_specs=pl.BlockSpec(memory_space=pltpu.SMEM),
  )
  dot = finish(hist, csum)
  return dot[0, 0]
